# baseline (speedup 1.0000x reference)
import jax
import jax.numpy as jnp
from jax import lax
from jax.experimental import pallas as pl
from jax.experimental.pallas import tpu as pltpu
from jax.experimental.pallas import tpu_sc as plsc

B = 256
N_IN = 10000
N_OUT = 4096
N_CONN = 32
LANES = 16
GROUP = 2


def _make_sc_kernel(n_in, n_out, b, n_conn, num_cores, num_subcores,
                    interpret=False):
    bh = b // num_cores
    out_per_s = n_out // num_subcores
    nchunk = bh // LANES
    wrow = (n_conn + 4) * LANES
    n_in_per_s = n_in // num_subcores
    nsteps = out_per_s // GROUP
    grows = GROUP * n_conn
    mesh = plsc.VectorSubcoreMesh(core_axis_name="c", subcore_axis_name="s",
                                  num_cores=num_cores,
                                  num_subcores=num_subcores)

    def body(xh, edges, wbc, out, xs_v, edges_v, wb_v, rows_v, ob_v,
             semr0, semr1, semw0, semw1, semo0, semo1):
        half = lax.axis_index("c")
        sub = lax.axis_index("s")
        base = sub * out_per_s

        pltpu.sync_copy(xh.at[half, pl.ds(sub * n_in_per_s, n_in_per_s)],
                        xs_v.at[pl.ds(sub * n_in_per_s, n_in_per_s)])
        pltpu.sync_copy(edges.at[pl.ds(base * n_conn, out_per_s * n_conn)],
                        edges_v)
        plsc.subcore_barrier()

        def gather(g, buf, semr, semw):
            idx = edges_v.at[pl.ds(g * grows, grows)]
            rows = pltpu.make_async_copy(xs_v.at[idx], rows_v.at[buf], semr)
            wts = pltpu.make_async_copy(
                wbc.at[pl.ds((base + g * GROUP) * wrow, GROUP * wrow)],
                wb_v.at[buf], semw)
            return rows, wts

        def out_copy(g, buf, semo):
            dst = out.at[pl.ds((half * n_out + base + g * GROUP) * bh,
                               GROUP * bh)]
            return pltpu.make_async_copy(ob_v.at[buf], dst, semo)

        def compute(g, buf):
            for j in range(GROUP):
                bias_vec = wb_v[buf, pl.ds(j * wrow + n_conn * LANES, LANES)]
                accs = [bias_vec] * nchunk
                for c in range(n_conn):
                    w = wb_v[buf, pl.ds(j * wrow + c * LANES, LANES)]
                    for k in range(nchunk):
                        accs[k] = (accs[k]
                                   + rows_v[buf, j * n_conn + c,
                                            pl.ds(k * LANES, LANES)] * w)
                for k in range(nchunk):
                    ob_v[buf, pl.ds(j * bh + k * LANES, LANES)] = accs[k]

        def step(g, buf, semr, semw, semo):
            r, w = gather(g, buf, semr, semw)
            r.wait()
            w.wait()

            @pl.when(g >= 2)
            def _():
                out_copy(g - 2, buf, semo).wait()

            compute(g, buf)
            out_copy(g, buf, semo).start()

            @pl.when(g + 2 < nsteps)
            def _():
                rn, wn = gather(g + 2, buf, semr, semw)
                rn.start()
                wn.start()

        r0, w0 = gather(0, 0, semr0, semw0)
        r1, w1 = gather(1, 1, semr1, semw1)
        r0.start()
        w0.start()
        r1.start()
        w1.start()

        def outer(g2, carry):
            g = g2 * 2
            step(g, 0, semr0, semw0, semo0)
            step(g + 1, 1, semr1, semw1, semo1)
            return carry

        lax.fori_loop(0, nsteps // 2, outer, 0)
        out_copy(nsteps - 2, 0, semo0).wait()
        out_copy(nsteps - 1, 1, semo1).wait()

    return pl.kernel(
        body,
        out_type=jax.ShapeDtypeStruct((num_cores * n_out * bh,), jnp.float32),
        mesh=mesh,
        scratch_types=[
            pltpu.VMEM_SHARED((n_in, bh), jnp.float32),
            pltpu.VMEM((out_per_s * n_conn,), jnp.int32),
            pltpu.VMEM((2, GROUP * wrow), jnp.float32),
            pltpu.VMEM((2, grows, bh), jnp.float32),
            pltpu.VMEM((2, GROUP * bh), jnp.float32),
            pltpu.SemaphoreType.DMA,
            pltpu.SemaphoreType.DMA,
            pltpu.SemaphoreType.DMA,
            pltpu.SemaphoreType.DMA,
            pltpu.SemaphoreType.DMA,
            pltpu.SemaphoreType.DMA,
        ],
        interpret=interpret,
    )


@jax.jit
def kernel(x, edges, kernel, bias):
    nc = 2
    bh = B // nc
    n_in_pad = ((N_IN + 127) // 128) * 128
    xh = x.T.reshape(N_IN, nc, bh).transpose(1, 0, 2)
    xh = jnp.pad(xh, ((0, 0), (0, n_in_pad - N_IN), (0, 0)))
    wbc = jnp.concatenate(
        [jnp.broadcast_to(kernel[:, :, None], (N_OUT, N_CONN, LANES)),
         jnp.broadcast_to(bias[:, None, None], (N_OUT, 1, LANES)),
         jnp.zeros((N_OUT, 3, LANES), jnp.float32)],
        axis=1).reshape(-1)
    sc = _make_sc_kernel(n_in_pad, N_OUT, B, N_CONN, nc, 16)
    yflat = sc(xh, edges.reshape(-1), wbc)
    return yflat.reshape(nc, N_OUT, bh).transpose(0, 2, 1).reshape(B, N_OUT)


# speedup vs baseline: 1.3613x; 1.3613x over previous
import jax
import jax.numpy as jnp
from jax import lax
from jax.experimental import pallas as pl
from jax.experimental.pallas import tpu as pltpu
from jax.experimental.pallas import tpu_sc as plsc

B = 256
N_IN = 10000
N_OUT = 4096
N_CONN = 32
LANES = 16
NBUF = 4
WREC = 40


def _make_sc_kernel(n_in, n_out, b, n_conn, num_cores, num_subcores,
                    interpret=False):
    nw = num_cores * num_subcores
    rows_per_w = b // nw
    ngroups = n_out // LANES
    erec = n_conn * LANES
    wrec = WREC * LANES
    mesh = plsc.VectorSubcoreMesh(core_axis_name="c", subcore_axis_name="s",
                                  num_cores=num_cores,
                                  num_subcores=num_subcores)

    def body(xf, et, wt, out, xr_v, et_v, wt_v, out_v, *sems):
        wid = lax.axis_index("s") * num_cores + lax.axis_index("c")
        row0 = wid * rows_per_w
        pltpu.sync_copy(xf.at[pl.ds(row0 * n_in, rows_per_w * n_in)], xr_v)

        def streams(g, buf, sem):
            e = pltpu.make_async_copy(et.at[pl.ds(g * erec, erec)],
                                      et_v.at[buf], sem)
            w = pltpu.make_async_copy(wt.at[pl.ds(g * wrec, wrec)],
                                      wt_v.at[buf], sem)
            return e, w

        def start(g, buf, sem):
            e, w = streams(g, buf, sem)
            e.start()
            w.start()

        def wait(g, buf, sem):
            e, w = streams(g, buf, sem)
            e.wait()
            w.wait()

        def compute(g, buf):
            bias_vec = wt_v[buf, pl.ds(n_conn * LANES, LANES)]
            accs = [bias_vec] * rows_per_w
            for c in range(n_conn):
                idx = et_v[buf, pl.ds(c * LANES, LANES)]
                w = wt_v[buf, pl.ds(c * LANES, LANES)]
                for r in range(rows_per_w):
                    vals = plsc.load_gather(xr_v, [idx + (r * n_in)])
                    accs[r] = accs[r] + vals * w
            for r in range(rows_per_w):
                out_v[pl.ds(r * n_out + g * LANES, LANES)] = accs[r]

        for j in range(NBUF):
            start(j, j, sems[j])

        def outer(i, carry):
            g0 = i * NBUF
            for j in range(NBUF):
                g = g0 + j
                wait(g, j, sems[j])
                compute(g, j)

                @pl.when(g + NBUF < ngroups)
                def _():
                    start(g + NBUF, j, sems[j])

            return carry

        lax.fori_loop(0, ngroups // NBUF, outer, 0)
        pltpu.sync_copy(out_v, out.at[pl.ds(row0 * n_out, rows_per_w * n_out)])

    return pl.kernel(
        body,
        out_type=jax.ShapeDtypeStruct((b * n_out,), jnp.float32),
        mesh=mesh,
        scratch_types=[
            pltpu.VMEM((b // nw * n_in,), jnp.float32),
            pltpu.VMEM((NBUF, erec), jnp.int32),
            pltpu.VMEM((NBUF, wrec), jnp.float32),
            pltpu.VMEM((b // nw * n_out,), jnp.float32),
        ] + [pltpu.SemaphoreType.DMA] * NBUF,
        compiler_params=pltpu.CompilerParams(needs_layout_passes=False),
        interpret=interpret,
    )


@jax.jit
def kernel(x, edges, kernel, bias):
    ng = N_OUT // LANES
    et = edges.reshape(ng, LANES, N_CONN).transpose(0, 2, 1).reshape(-1)
    wt = jnp.concatenate(
        [kernel.reshape(ng, LANES, N_CONN).transpose(0, 2, 1),
         bias.reshape(ng, 1, LANES),
         jnp.zeros((ng, WREC - N_CONN - 1, LANES), jnp.float32)],
        axis=1).reshape(-1)
    sc = _make_sc_kernel(N_IN, N_OUT, B, N_CONN, 2, 16)
    return sc(x.reshape(-1), et, wt).reshape(B, N_OUT)


# speedup vs baseline: 2.0607x; 1.5137x over previous
import jax
import jax.numpy as jnp
from jax import lax
from jax.experimental import pallas as pl
from jax.experimental.pallas import tpu as pltpu
from jax.experimental.pallas import tpu_sc as plsc

B = 256
N_IN = 10000
N_OUT = 4096
N_CONN = 32
LANES = 16
NBUF = 4
WREC = 40


def _make_sc_kernel(n_in, n_out, b, n_conn, num_cores, num_subcores,
                    interpret=False):
    nw = num_cores * num_subcores
    rows_per_w = b // nw
    ngroups = n_out // LANES
    erec = n_conn * LANES
    wrec = WREC * LANES
    mesh = plsc.VectorSubcoreMesh(core_axis_name="c", subcore_axis_name="s",
                                  num_cores=num_cores,
                                  num_subcores=num_subcores)

    def body(xf, et, wt, out, xr_v, et_v, wt_v, out_v, *sems):
        wid = lax.axis_index("s") * num_cores + lax.axis_index("c")
        row0 = wid * rows_per_w
        pltpu.sync_copy(xf.at[pl.ds(row0 * n_in, rows_per_w * n_in)], xr_v)

        def streams(g, buf, sem):
            e = pltpu.make_async_copy(et.at[pl.ds(g * erec, erec)],
                                      et_v.at[buf], sem)
            w = pltpu.make_async_copy(wt.at[pl.ds(g * wrec, wrec)],
                                      wt_v.at[buf], sem)
            return e, w

        def start(g, buf, sem):
            e, w = streams(g, buf, sem)
            e.start()
            w.start()

        def wait(g, buf, sem):
            e, w = streams(g, buf, sem)
            e.wait()
            w.wait()

        CSTEP = 4

        def compute(g, buf):
            bias_vec = wt_v[buf, pl.ds(n_conn * LANES, LANES)]

            def cbody(cc, accs):
                off = cc * (CSTEP * LANES)
                accs = list(accs)
                for k in range(CSTEP):
                    idx = et_v[buf, pl.ds(off + k * LANES, LANES)]
                    w = wt_v[buf, pl.ds(off + k * LANES, LANES)]
                    for r in range(rows_per_w):
                        vals = plsc.load_gather(xr_v, [idx + (r * n_in)])
                        accs[r] = accs[r] + vals * w
                return tuple(accs)

            accs = lax.fori_loop(0, n_conn // CSTEP, cbody,
                                 (bias_vec,) * rows_per_w)
            for r in range(rows_per_w):
                out_v[pl.ds(r * n_out + g * LANES, LANES)] = accs[r]

        for j in range(NBUF):
            start(j, j, sems[j])

        def outer(i, carry):
            g0 = i * NBUF
            for j in range(NBUF):
                g = g0 + j
                wait(g, j, sems[j])
                compute(g, j)
                jp = (j - 1) % NBUF
                gn = g - 1 + NBUF

                @pl.when(jnp.logical_and(g >= 1, gn < ngroups))
                def _():
                    start(gn, jp, sems[jp])

            return carry

        lax.fori_loop(0, ngroups // NBUF, outer, 0)
        pltpu.sync_copy(out_v, out.at[pl.ds(row0 * n_out, rows_per_w * n_out)])

    return pl.kernel(
        body,
        out_type=jax.ShapeDtypeStruct((b * n_out,), jnp.float32),
        mesh=mesh,
        scratch_types=[
            pltpu.VMEM((b // nw * n_in,), jnp.float32),
            pltpu.VMEM((NBUF, erec), jnp.int32),
            pltpu.VMEM((NBUF, wrec), jnp.float32),
            pltpu.VMEM((b // nw * n_out,), jnp.float32),
        ] + [pltpu.SemaphoreType.DMA] * NBUF,
        compiler_params=pltpu.CompilerParams(needs_layout_passes=False),
        interpret=interpret,
    )


@jax.jit
def kernel(x, edges, kernel, bias):
    ng = N_OUT // LANES
    et = edges.reshape(ng, LANES, N_CONN).transpose(0, 2, 1).reshape(-1)
    wt = jnp.concatenate(
        [kernel.reshape(ng, LANES, N_CONN).transpose(0, 2, 1),
         bias.reshape(ng, 1, LANES),
         jnp.zeros((ng, WREC - N_CONN - 1, LANES), jnp.float32)],
        axis=1).reshape(-1)
    sc = _make_sc_kernel(N_IN, N_OUT, B, N_CONN, 2, 16)
    return sc(x.reshape(-1), et, wt).reshape(B, N_OUT)
